# baseline (speedup 1.0000x reference)
import functools
import jax
import jax.numpy as jnp
from jax import lax
from jax.experimental import pallas as pl
from jax.experimental.pallas import tpu as pltpu
from jax.experimental.pallas import tpu_sc as plsc

_B, _R, _F, _D = 4, 1024, 100, 128
_FP = 128
_NW = 32
_RPW = (_B * _R) // _NW
_CH = 4
_NCH = _RPW // _CH
_MAGIC = 12582912.0


def _moments_body(tr_ref, x_ref, mean_ref, istd_ref):
    tr = tr_ref[0]
    cnt = jnp.maximum(tr.astype(jnp.float32), 1.0)
    rmask = (lax.broadcasted_iota(jnp.int32, (_R, _F), 0) < tr).astype(jnp.float32)
    for b in range(_B):
        xb = x_ref[b]
        mean = jnp.sum(xb * rmask, axis=0, keepdims=True) / cnt
        var = jnp.sum(rmask * (xb - mean) ** 2, axis=0, keepdims=True) / cnt
        istd = 1.0 / jnp.maximum(jnp.sqrt(var), 1e-20)
        mean_ref[b, :] = mean[0]
        istd_ref[b, :] = istd[0]


def _sc_body(xp_hbm, mean_hbm, istd_hbm, fic_hbm, card_hbm, tab_hbm, wb_hbm,
             out_hbm, tab_v, wb_v, par_v, ms_v, x_v, stage_v):
    wid = lax.axis_index("s") * 2 + lax.axis_index("c")
    base = wid * _RPW
    bidx = wid // (_NW // _B)
    pltpu.sync_copy(tab_hbm, tab_v)
    pltpu.sync_copy(wb_hbm, wb_v)
    pltpu.sync_copy(fic_hbm, par_v.at[pl.ds(0, _FP)])
    pltpu.sync_copy(card_hbm, par_v.at[pl.ds(_FP, _FP)])
    pltpu.sync_copy(mean_hbm.at[pl.ds(bidx * _FP, _FP)], ms_v.at[pl.ds(0, _FP)])
    pltpu.sync_copy(istd_hbm.at[pl.ds(bidx * _FP, _FP)], ms_v.at[pl.ds(_FP, _FP)])

    wv = [wb_v[pl.ds(jj * 16, 16)] for jj in range(8)]
    bv = [wb_v[pl.ds(_D + jj * 16, 16)] for jj in range(8)]

    def chunk_body(ci, carry):
        row0 = base + ci * _CH
        pltpu.sync_copy(xp_hbm.at[pl.ds(row0 * _FP, _CH * _FP)], x_v)

        def row_body(c, carry2):
            jb = c * _FP
            ob = c * (_F * _D)
            for g in range(7):
                lanes = 16 if g < 6 else _F - 6 * 16
                xv = x_v[pl.ds(jb + g * 16, 16)]
                mnv = ms_v[pl.ds(g * 16, 16)]
                isdv = ms_v[pl.ds(_FP + g * 16, 16)]
                ficv = par_v[pl.ds(g * 16, 16)]
                cardv = par_v[pl.ds(_FP + g * 16, 16)]
                rawv = (xv + _MAGIC) - _MAGIC
                invalv = (rawv < 0.0) | (rawv >= cardv) | (rawv >= 64.0)
                idzv = (jnp.where(invalv, 0.0, rawv + 1.0) * float(_D)
                        ).astype(jnp.int32)
                xnv = jnp.minimum(jnp.maximum((xv - mnv) * isdv, -100.0), 100.0)
                for l in range(lanes):
                    fi = ficv[l]
                    tb = idzv[l]
                    xs = xnv[l]
                    oe = ob + (g * 16 + l) * _D

                    def cat_fn(tb=tb, oe=oe):
                        for jj in range(8):
                            stage_v[pl.ds(oe + jj * 16, 16)] = (
                                tab_v[pl.ds(tb + jj * 16, 16)])

                    def cont_fn(xs=xs, oe=oe):
                        for jj in range(8):
                            stage_v[pl.ds(oe + jj * 16, 16)] = wv[jj] * xs + bv[jj]

                    lax.cond(fi != 0.0, cat_fn, cont_fn)
            return carry2

        lax.fori_loop(0, _CH, row_body, 0)
        pltpu.sync_copy(stage_v, out_hbm.at[pl.ds(row0 * (_F * _D), _CH * _F * _D)])
        return carry

    lax.fori_loop(0, _NCH, chunk_body, 0)


def kernel(x, train_test_split_index, feature_is_categorical,
           feature_cardinalities, linear_W, linear_b, emb_table,
           cont_type, cat_type):
    tr = jnp.clip(jnp.asarray(train_test_split_index, jnp.int32).reshape(-1)[:1],
                  0, _R)
    mean, istd = pl.pallas_call(
        _moments_body,
        in_specs=[pl.BlockSpec(memory_space=pltpu.SMEM),
                  pl.BlockSpec((_B, _R, _F), lambda: (0, 0, 0))],
        out_specs=[pl.BlockSpec((_B, _F), lambda: (0, 0)),
                   pl.BlockSpec((_B, _F), lambda: (0, 0))],
        out_shape=[jax.ShapeDtypeStruct((_B, _F), jnp.float32),
                   jax.ShapeDtypeStruct((_B, _F), jnp.float32)],
    )(tr, x)

    pad = ((0, 0), (0, _FP - _F))
    mean_p = jnp.pad(mean, pad).reshape(_B * _FP)
    istd_p = jnp.pad(istd, pad).reshape(_B * _FP)
    fic_p = jnp.pad(feature_is_categorical.astype(jnp.float32), (0, _FP - _F))
    card_p = jnp.pad(
        jnp.maximum(feature_cardinalities.astype(jnp.int32), 1).astype(jnp.float32),
        (0, _FP - _F))
    tab = (emb_table + cat_type.reshape(1, _D)).reshape(65 * _D)
    wb = jnp.concatenate([linear_W[:, 0], linear_b + cont_type.reshape(_D)])
    xp = jnp.pad(x.reshape(_B * _R, _F), pad).reshape(_B * _R * _FP)

    mesh = plsc.VectorSubcoreMesh(core_axis_name="c", subcore_axis_name="s")
    sck = functools.partial(
        pl.kernel, mesh=mesh,
        out_type=jax.ShapeDtypeStruct((_B * _R * _F * _D,), jnp.float32),
        scratch_types=[
            pltpu.VMEM((65 * _D,), jnp.float32),
            pltpu.VMEM((2 * _D,), jnp.float32),
            pltpu.VMEM((2 * _FP,), jnp.float32),
            pltpu.VMEM((2 * _FP,), jnp.float32),
            pltpu.VMEM((_CH * _FP,), jnp.float32),
            pltpu.VMEM((_CH * _F * _D,), jnp.float32),
        ],
    )(_sc_body)
    out = sck(xp, mean_p, istd_p, fic_p, card_p, tab, wb)
    return out.reshape(_B, _R, _F, _D)


# speedup vs baseline: 5.0095x; 5.0095x over previous
import jax
import jax.numpy as jnp
from jax import lax
from jax.experimental import pallas as pl
from jax.experimental.pallas import tpu as pltpu

_B, _R, _F, _D = 4, 1024, 100, 128
_K = 128
_RT = 256


def _moments_body(tr_ref, x_ref, mean_ref, istd_ref):
    tr = tr_ref[0]
    cnt = jnp.maximum(tr.astype(jnp.float32), 1.0)
    rmask = (lax.broadcasted_iota(jnp.int32, (_R, _F), 0) < tr).astype(jnp.float32)
    for b in range(_B):
        xb = x_ref[b]
        mean = jnp.sum(xb * rmask, axis=0, keepdims=True) / cnt
        var = jnp.sum(rmask * (xb - mean) ** 2, axis=0, keepdims=True) / cnt
        istd = 1.0 / jnp.maximum(jnp.sqrt(var), 1e-20)
        mean_ref[b, :] = mean[0]
        istd_ref[b, :] = istd[0]


def _encode_body(x_ref, mean_ref, istd_ref, card_ref, ficl_ref,
                 rhs_ref, nb_ref, out_ref):
    xs = x_ref[0]
    xn = jnp.clip((xs - mean_ref[0]) * istd_ref[0], -100.0, 100.0)
    raw = jnp.round(xs)
    invalid = (raw < 0.0) | (raw >= card_ref[0]) | (raw >= 64.0)
    z = jnp.where(invalid, 0.0, raw + 1.0)
    ficm = ficl_ref[0] != 0.0
    q = jnp.where(ficm, z, 65.0).astype(jnp.bfloat16)
    s = jnp.where(ficm, 1.0, xn).astype(jnp.bfloat16)
    q3 = q[:, :, None]
    s3 = s[:, :, None]
    kv = lax.broadcasted_iota(jnp.int32, (1, 1, _K), 2).astype(jnp.bfloat16)
    g = jnp.where(q3 == kv, s3, jnp.bfloat16(0.0))
    mm = lax.dot_general(g, rhs_ref[...], (((2,), (0,)), ((), ())),
                         preferred_element_type=jnp.float32)
    out_ref[0] = mm + nb_ref[0]


def kernel(x, train_test_split_index, feature_is_categorical,
           feature_cardinalities, linear_W, linear_b, emb_table,
           cont_type, cat_type):
    tr = jnp.clip(jnp.asarray(train_test_split_index, jnp.int32).reshape(-1)[:1],
                  0, _R)
    mean, istd = pl.pallas_call(
        _moments_body,
        in_specs=[pl.BlockSpec(memory_space=pltpu.SMEM),
                  pl.BlockSpec((_B, _R, _F), lambda: (0, 0, 0))],
        out_specs=[pl.BlockSpec((_B, _F), lambda: (0, 0)),
                   pl.BlockSpec((_B, _F), lambda: (0, 0))],
        out_shape=[jax.ShapeDtypeStruct((_B, _F), jnp.float32),
                   jax.ShapeDtypeStruct((_B, _F), jnp.float32)],
    )(tr, x)

    fic_f = feature_is_categorical.astype(jnp.float32)
    card_f = jnp.maximum(feature_cardinalities.astype(jnp.int32), 1).astype(jnp.float32)
    w_row = linear_W[:, 0]
    b2 = linear_b + cont_type.reshape(_D)
    table2 = emb_table + cat_type.reshape(1, _D)
    rhs = jnp.concatenate(
        [table2, w_row[None, :], jnp.zeros((_K - 66, _D), jnp.float32)],
        axis=0).astype(jnp.bfloat16)
    nb = (1.0 - fic_f)[:, None] * b2[None, :]

    out = pl.pallas_call(
        _encode_body,
        grid=(_B, _R // _RT),
        in_specs=[
            pl.BlockSpec((1, _RT, _F), lambda b, r: (b, r, 0)),
            pl.BlockSpec((1, 1, _F), lambda b, r: (b, 0, 0)),
            pl.BlockSpec((1, 1, _F), lambda b, r: (b, 0, 0)),
            pl.BlockSpec((1, 1, _F), lambda b, r: (0, 0, 0)),
            pl.BlockSpec((1, 1, _F), lambda b, r: (0, 0, 0)),
            pl.BlockSpec((_K, _D), lambda b, r: (0, 0)),
            pl.BlockSpec((1, _F, _D), lambda b, r: (0, 0, 0)),
        ],
        out_specs=pl.BlockSpec((1, _RT, _F, _D), lambda b, r: (b, r, 0, 0)),
        out_shape=jax.ShapeDtypeStruct((_B, _R, _F, _D), jnp.float32),
        compiler_params=pltpu.CompilerParams(
            dimension_semantics=("parallel", "parallel")),
    )(x,
      mean.reshape(_B, 1, _F), istd.reshape(_B, 1, _F),
      card_f.reshape(1, 1, _F), fic_f.reshape(1, 1, _F),
      rhs, nb.reshape(1, _F, _D))
    return out
